# baseline (speedup 1.0000x reference)
import functools

import jax
import jax.numpy as jnp
from jax import lax
from jax.experimental import pallas as pl
from jax.experimental.pallas import tpu as pltpu
from jax.experimental.pallas import tpu_sc as plsc

NC, NS = 2, 16
NW = NC * NS
LANES = 16

_SC_PARAMS = dict(
    compiler_params=pltpu.CompilerParams(needs_layout_passes=False))


def _make_entity_gather(n_ent, hid, n_pad):
    rows_per_w = n_pad // NW
    CH = 80
    n_ch = rows_per_w // CH

    mesh = plsc.VectorSubcoreMesh(core_axis_name="c", subcore_axis_name="s")

    @functools.partial(
        pl.kernel,
        out_type=jax.ShapeDtypeStruct((n_pad, hid), jnp.float32),
        mesh=mesh,
        scratch_types=[
            pltpu.VMEM((rows_per_w,), jnp.int32),
        ] + [pltpu.VMEM((CH, hid), jnp.float32) for _ in range(n_ch)]
          + [pltpu.SemaphoreType.DMA for _ in range(n_ch)]
          + [pltpu.SemaphoreType.DMA],
        **_SC_PARAMS,
    )
    def gather_kernel(table_hbm, ids_hbm, out_hbm, idx_v, *bufs_and_sems):
        rows = bufs_and_sems[:n_ch]
        sems = bufs_and_sems[n_ch:2 * n_ch]
        semw = bufs_and_sems[2 * n_ch]
        wid = lax.axis_index("c") * NS + lax.axis_index("s")
        base = wid * rows_per_w
        pltpu.sync_copy(ids_hbm.at[pl.ds(base, rows_per_w)], idx_v)
        cps = [pltpu.async_copy(
            table_hbm.at[idx_v.at[pl.ds(c * CH, CH)]], rows[c], sems[c])
            for c in range(n_ch)]
        for c in range(n_ch):
            cps[c].wait()
            pltpu.async_copy(rows[c], out_hbm.at[pl.ds(base + c * CH, CH)],
                             semw)
        for c in range(n_ch):
            pltpu.make_async_copy(
                rows[c], out_hbm.at[pl.ds(0, CH)], semw).wait()

    return gather_kernel


def _pack_bf16_pairs(x, hid):
    half = hid // 2
    lo = x[:, :half].astype(jnp.bfloat16).astype(jnp.float32)
    hi = x[:, half:].astype(jnp.bfloat16).astype(jnp.float32)
    lo_b = lax.bitcast_convert_type(lo, jnp.uint32) >> 16
    hi_b = lax.bitcast_convert_type(hi, jnp.uint32) & jnp.uint32(0xFFFF0000)
    return lax.bitcast_convert_type(lo_b | hi_b, jnp.int32)


def _tc_node_body(ids_ref, raw_ref, went_ref, bent_ref, nte_ref, w1_ref,
                  w3_ref, nt_ref, h_ref, t_ref):
    hid = raw_ref.shape[1]
    nt = jnp.dot(raw_ref[...], went_ref[...],
                 preferred_element_type=jnp.float32) + bent_ref[...]
    nt = jnp.where(ids_ref[...] == 0, nte_ref[...], nt)
    nt_ref[...] = nt
    h_ref[...] = _pack_bf16_pairs(
        jnp.dot(nt, w1_ref[...], preferred_element_type=jnp.float32), hid)
    t_ref[...] = _pack_bf16_pairs(
        jnp.dot(nt, w3_ref[...], preferred_element_type=jnp.float32), hid)


def _tc_rel_body(relt_ref, wrel_ref, brel_ref, w2_ref, bedge_ref,
                 q_ref, wq_ref, bq_ref, src2d_ref, ptr_ref,
                 rp_ref, qt_ref, eb_ref, ptrrow_ref):
    hid = relt_ref.shape[1]
    rel_tok = jnp.dot(relt_ref[...], wrel_ref[...],
                      preferred_element_type=jnp.float32) + brel_ref[...]
    rp_ref[...] = _pack_bf16_pairs(
        jnp.dot(rel_tok, w2_ref[...],
                preferred_element_type=jnp.float32) + bedge_ref[...], hid)
    qt_ref[...] = jnp.dot(q_ref[...], wq_ref[...],
                          preferred_element_type=jnp.float32) + bq_ref[...]
    src = src2d_ref[...]
    n_edges = src.size
    n_ptr = ptr_ref.shape[0]
    n_graphs = n_ptr - 1
    cnt = jnp.zeros(src.shape, jnp.int32)
    lane = lax.broadcasted_iota(jnp.int32, (1, 128), 1)
    row = jnp.where(lane == n_graphs, n_edges, jnp.zeros((1, 128), jnp.int32))
    for g in range(n_ptr):
        geq = (src >= ptr_ref[g]).astype(jnp.int32)
        cnt = cnt + geq
        if 1 <= g <= n_graphs - 1:
            row = jnp.where(lane == g, n_edges - jnp.sum(geq), row)
    eb_ref[...] = jnp.clip(cnt - 1, 0, n_graphs - 1)
    ptrrow_ref[...] = row


def _make_edge_kernel(n_nodes, n_rel, hid, n_edges):
    K = 40
    n_chunks = n_edges // K
    n_my = n_chunks // NW
    e_my = n_my * K
    half = hid // 2
    n_grp = half // LANES
    e_pad = ((e_my + LANES - 1) // LANES) * LANES
    n_gv = e_pad // LANES

    mesh = plsc.VectorSubcoreMesh(core_axis_name="c", subcore_axis_name="s")

    @functools.partial(
        pl.kernel,
        out_type=(
            jax.ShapeDtypeStruct((n_edges, hid), jnp.float32),
            jax.ShapeDtypeStruct((n_edges,), jnp.int32),
            jax.ShapeDtypeStruct((n_edges,), jnp.int32),
        ),
        mesh=mesh,
        scratch_types=[
            pltpu.VMEM((n_nodes,), jnp.int32),
            pltpu.VMEM((e_pad,), jnp.int32),
            pltpu.VMEM((e_pad,), jnp.int32),
            pltpu.VMEM((e_my,), jnp.int32),
            pltpu.VMEM((e_pad,), jnp.int32),
            pltpu.VMEM((e_pad,), jnp.int32),
            pltpu.VMEM((K, half), jnp.int32),
            pltpu.VMEM((K, half), jnp.int32),
            pltpu.VMEM((K, half), jnp.int32),
            pltpu.VMEM((K, half), jnp.int32),
            pltpu.VMEM((K, half), jnp.int32),
            pltpu.VMEM((K, half), jnp.int32),
            pltpu.VMEM((K, half), jnp.int32),
            pltpu.VMEM((K, half), jnp.int32),
            pltpu.VMEM((K, half), jnp.int32),
            pltpu.VMEM((K, hid), jnp.float32),
            pltpu.VMEM((K, hid), jnp.float32),
            pltpu.VMEM((K, hid), jnp.float32),
            pltpu.SemaphoreType.DMA,
            pltpu.SemaphoreType.DMA,
            pltpu.SemaphoreType.DMA,
            pltpu.SemaphoreType.DMA,
            pltpu.SemaphoreType.DMA,
            pltpu.SemaphoreType.DMA,
        ],
        **_SC_PARAMS,
    )
    def edge_kernel(h_hbm, t_hbm, rp_hbm, src_hbm, dst_hbm, rel_hbm, ngid_hbm,
                    out_hbm, heads_hbm, tails_hbm,
                    ngid_v, src_all, dst_all, rel_all, hbuf_all, tbuf_all,
                    h0, h1, h2, t0, t1, t2, r0, r1, r2, o0, o1, o2,
                    semg0, semg1, semg2, semw0, semw1, semw2):
        h_rows = (h0, h1, h2)
        t_rows = (t0, t1, t2)
        r_rows = (r0, r1, r2)
        out_buf = (o0, o1, o2)
        semg = (semg0, semg1, semg2)
        semw = (semw0, semw1, semw2)
        wid = lax.axis_index("c") * NS + lax.axis_index("s")
        ebase = wid * e_my

        zeros16 = jnp.zeros((LANES,), jnp.int32)
        src_all[pl.ds(e_pad - LANES, LANES)] = zeros16
        dst_all[pl.ds(e_pad - LANES, LANES)] = zeros16
        pltpu.sync_copy(src_hbm.at[pl.ds(ebase, e_my)],
                        src_all.at[pl.ds(0, e_my)])
        pltpu.sync_copy(dst_hbm.at[pl.ds(ebase, e_my)],
                        dst_all.at[pl.ds(0, e_my)])
        pltpu.sync_copy(rel_hbm.at[pl.ds(ebase, e_my)], rel_all)
        pltpu.sync_copy(ngid_hbm, ngid_v)

        def fire(c, b):
            lb = c * K
            pltpu.async_copy(h_hbm.at[src_all.at[pl.ds(lb, K)]],
                             h_rows[b], semg[b])
            pltpu.async_copy(t_hbm.at[dst_all.at[pl.ds(lb, K)]],
                             t_rows[b], semg[b])
            pltpu.async_copy(rp_hbm.at[rel_all.at[pl.ds(lb, K)]],
                             r_rows[b], semg[b])

        fire(0, 0)
        fire(1, 1)

        @plsc.parallel_loop(0, n_gv, unroll=4)
        def _gid(j):
            sl = pl.ds(j * LANES, LANES)
            hbuf_all[sl] = plsc.load_gather(ngid_v, [src_all[sl]])
            tbuf_all[sl] = plsc.load_gather(ngid_v, [dst_all[sl]])

        pltpu.async_copy(hbuf_all.at[pl.ds(0, e_my)],
                         heads_hbm.at[pl.ds(ebase, e_my)], semw0)
        pltpu.async_copy(tbuf_all.at[pl.ds(0, e_my)],
                         tails_hbm.at[pl.ds(ebase, e_my)], semw1)
        pltpu.make_async_copy(hbuf_all.at[pl.ds(0, e_my)],
                              heads_hbm.at[pl.ds(0, e_my)], semw0).wait()
        pltpu.make_async_copy(tbuf_all.at[pl.ds(0, e_my)],
                              tails_hbm.at[pl.ds(0, e_my)], semw1).wait()

        @pl.loop(0, n_my, step=3)
        def _trip(lc):
            for b in range(3):
                c = lc + b
                live = (c < n_my) if b else None

                def section():
                    @pl.when(c + 2 < n_my)
                    def _():
                        fire(c + 2, (b + 2) % 3)
                    for dst in (h_rows[b], t_rows[b], r_rows[b]):
                        pltpu.make_async_copy(
                            h_hbm.at[pl.ds(0, K)], dst, semg[b]).wait()
                    @pl.when(c >= 3)
                    def _():
                        pltpu.make_async_copy(
                            out_buf[b], out_hbm.at[pl.ds(0, K)],
                            semw[b]).wait()

                    mask_hi = jnp.int32(-65536)

                    @plsc.parallel_loop(0, K, unroll=4)
                    def _row(r):
                        for m in range(n_grp):
                            sl = pl.ds(m * LANES, LANES)
                            wh = h_rows[b][r, sl]
                            wt = t_rows[b][r, sl]
                            wr = r_rows[b][r, sl]
                            lo = (plsc.bitcast(wh << 16, jnp.float32)
                                  + plsc.bitcast(wt << 16, jnp.float32)
                                  + plsc.bitcast(wr << 16, jnp.float32))
                            hi = (plsc.bitcast(wh & mask_hi, jnp.float32)
                                  + plsc.bitcast(wt & mask_hi, jnp.float32)
                                  + plsc.bitcast(wr & mask_hi, jnp.float32))
                            out_buf[b][r, sl] = jnp.maximum(lo, 0.0)
                            out_buf[b][r, pl.ds(half + m * LANES, LANES)] = (
                                jnp.maximum(hi, 0.0))

                    pltpu.async_copy(
                        out_buf[b],
                        out_hbm.at[pl.ds(ebase + c * K, K)], semw[b])

                if live is None:
                    section()
                else:
                    pl.when(live)(section)

        for b in range(3):
            pltpu.make_async_copy(out_buf[b], out_hbm.at[pl.ds(0, K)],
                                  semw[b]).wait()

    return edge_kernel


def kernel(node_embedding_ids, node_global_ids, edge_index, edge_relations,
           question_emb, node_ptr, entity_table, relation_table, non_text_emb,
           W_ent, b_ent, W_rel, b_rel, W_q, b_q, W_edge, b_edge):
    n_nodes = node_embedding_ids.shape[0]
    n_edges = edge_relations.shape[0]
    hid = entity_table.shape[1]
    n_ent = entity_table.shape[0]
    n_rel = relation_table.shape[0]
    n_graphs = node_ptr.shape[0] - 1

    src = edge_index[0]
    dst = edge_index[1]

    W1 = W_edge[0 * hid:1 * hid]
    W2 = W_edge[1 * hid:2 * hid]
    W3 = W_edge[2 * hid:3 * hid]
    ids_col = node_embedding_ids.reshape(n_nodes, 1)
    src2d = src.reshape(n_edges // 128, 128)
    row2 = lambda v: v.reshape(1, hid)

    full = lambda shape: pl.BlockSpec(shape, lambda i: (0, 0))

    n_pad = ((n_nodes + NW * 80 - 1) // (NW * 80)) * (NW * 80)
    ids_pad = jnp.pad(node_embedding_ids, (0, n_pad - n_nodes))
    node_raw_pad = _make_entity_gather(n_ent, hid, n_pad)(entity_table, ids_pad)

    Rp, question_tokens, eb2d, ptr_row = pl.pallas_call(
        _tc_rel_body,
        grid=(1,),
        in_specs=[
            full((n_rel, hid)), full((hid, hid)), full((1, hid)),
            full((hid, hid)), full((1, hid)),
            full((question_emb.shape[0], hid)), full((hid, hid)), full((1, hid)),
            full((n_edges // 128, 128)),
            pl.BlockSpec(memory_space=pltpu.SMEM),
        ],
        out_specs=[
            full((n_rel, hid // 2)),
            full((question_emb.shape[0], hid)),
            full((n_edges // 128, 128)),
            full((1, 128)),
        ],
        out_shape=[
            jax.ShapeDtypeStruct((n_rel, hid // 2), jnp.int32),
            jax.ShapeDtypeStruct((question_emb.shape[0], hid), jnp.float32),
            jax.ShapeDtypeStruct((n_edges // 128, 128), jnp.int32),
            jax.ShapeDtypeStruct((1, 128), jnp.int32),
        ],
    )(relation_table, W_rel, row2(b_rel), W2, row2(b_edge),
      question_emb, W_q, row2(b_q), src2d, node_ptr)

    BLK = 2000
    n_blk = n_nodes // BLK
    blk_node = pl.BlockSpec((BLK, hid), lambda i: (i, 0))
    blk_pack = pl.BlockSpec((BLK, hid // 2), lambda i: (i, 0))

    node_tokens, H, T = pl.pallas_call(
        _tc_node_body,
        grid=(n_blk,),
        in_specs=[
            pl.BlockSpec((BLK, 1), lambda i: (i, 0)),
            blk_node,
            full((hid, hid)), full((1, hid)), full((1, hid)),
            full((hid, hid)), full((hid, hid)),
        ],
        out_specs=[blk_node, blk_pack, blk_pack],
        out_shape=[
            jax.ShapeDtypeStruct((n_nodes, hid), jnp.float32),
            jax.ShapeDtypeStruct((n_nodes, hid // 2), jnp.int32),
            jax.ShapeDtypeStruct((n_nodes, hid // 2), jnp.int32),
        ],
    )(ids_col, node_raw_pad, W_ent, row2(b_ent), row2(non_text_emb), W1, W3)

    edge_tokens, heads_global, tails_global = _make_edge_kernel(
        n_nodes, n_rel, hid, n_edges)(H, T, Rp, src, dst, edge_relations,
                                      node_global_ids)

    edge_batch = eb2d.reshape(n_edges)
    edge_ptr = ptr_row[0, :n_graphs + 1]
    return (edge_tokens, node_tokens, question_tokens, heads_global,
            tails_global, edge_batch, edge_ptr)


# speedup vs baseline: 1.0093x; 1.0093x over previous
import functools

import jax
import jax.numpy as jnp
from jax import lax
from jax.experimental import pallas as pl
from jax.experimental.pallas import tpu as pltpu
from jax.experimental.pallas import tpu_sc as plsc

NC, NS = 2, 16
NW = NC * NS
LANES = 16

_SC_PARAMS = dict(
    compiler_params=pltpu.CompilerParams(needs_layout_passes=False))


def _make_entity_gather(n_ent, hid, n_pad):
    rows_per_w = n_pad // NW
    CH = 80
    n_ch = rows_per_w // CH

    mesh = plsc.VectorSubcoreMesh(core_axis_name="c", subcore_axis_name="s")

    @functools.partial(
        pl.kernel,
        out_type=jax.ShapeDtypeStruct((n_pad, hid), jnp.float32),
        mesh=mesh,
        scratch_types=[
            pltpu.VMEM((rows_per_w,), jnp.int32),
        ] + [pltpu.VMEM((CH, hid), jnp.float32) for _ in range(n_ch)]
          + [pltpu.SemaphoreType.DMA for _ in range(n_ch)]
          + [pltpu.SemaphoreType.DMA],
        **_SC_PARAMS,
    )
    def gather_kernel(table_hbm, ids_hbm, out_hbm, idx_v, *bufs_and_sems):
        rows = bufs_and_sems[:n_ch]
        sems = bufs_and_sems[n_ch:2 * n_ch]
        semw = bufs_and_sems[2 * n_ch]
        wid = lax.axis_index("c") * NS + lax.axis_index("s")
        base = wid * rows_per_w
        pltpu.sync_copy(ids_hbm.at[pl.ds(base, rows_per_w)], idx_v)
        cps = [pltpu.async_copy(
            table_hbm.at[idx_v.at[pl.ds(c * CH, CH)]], rows[c], sems[c])
            for c in range(n_ch)]
        for c in range(n_ch):
            cps[c].wait()
            pltpu.async_copy(rows[c], out_hbm.at[pl.ds(base + c * CH, CH)],
                             semw)
        for c in range(n_ch):
            pltpu.make_async_copy(
                rows[c], out_hbm.at[pl.ds(0, CH)], semw).wait()

    return gather_kernel


def _pack_bf16_pairs(x, hid):
    half = hid // 2
    lo = x[:, :half].astype(jnp.bfloat16).astype(jnp.float32)
    hi = x[:, half:].astype(jnp.bfloat16).astype(jnp.float32)
    lo_b = lax.bitcast_convert_type(lo, jnp.uint32) >> 16
    hi_b = lax.bitcast_convert_type(hi, jnp.uint32) & jnp.uint32(0xFFFF0000)
    return lax.bitcast_convert_type(lo_b | hi_b, jnp.int32)


def _tc_node_body(ids_ref, raw_ref, went_ref, bent_ref, nte_ref, w1_ref,
                  w3_ref, nt_ref, h_ref, t_ref):
    hid = raw_ref.shape[1]
    nt = jnp.dot(raw_ref[...], went_ref[...],
                 preferred_element_type=jnp.float32) + bent_ref[...]
    nt = jnp.where(ids_ref[...] == 0, nte_ref[...], nt)
    nt_ref[...] = nt
    h_ref[...] = _pack_bf16_pairs(
        jnp.dot(nt, w1_ref[...], preferred_element_type=jnp.float32), hid)
    t_ref[...] = _pack_bf16_pairs(
        jnp.dot(nt, w3_ref[...], preferred_element_type=jnp.float32), hid)


def _tc_rel_body(relt_ref, wrel_ref, brel_ref, w2_ref, bedge_ref,
                 q_ref, wq_ref, bq_ref, src2d_ref, ptr_ref,
                 rp_ref, qt_ref, eb_ref, ptrrow_ref):
    hid = relt_ref.shape[1]
    rel_tok = jnp.dot(relt_ref[...], wrel_ref[...],
                      preferred_element_type=jnp.float32) + brel_ref[...]
    rp_ref[...] = _pack_bf16_pairs(
        jnp.dot(rel_tok, w2_ref[...],
                preferred_element_type=jnp.float32) + bedge_ref[...], hid)
    qt_ref[...] = jnp.dot(q_ref[...], wq_ref[...],
                          preferred_element_type=jnp.float32) + bq_ref[...]
    src = src2d_ref[...]
    n_edges = src.size
    n_ptr = ptr_ref.shape[0]
    n_graphs = n_ptr - 1
    cnt = jnp.zeros(src.shape, jnp.int32)
    lane = lax.broadcasted_iota(jnp.int32, (1, 128), 1)
    row = jnp.where(lane == n_graphs, n_edges, jnp.zeros((1, 128), jnp.int32))
    for g in range(n_ptr):
        geq = (src >= ptr_ref[g]).astype(jnp.int32)
        cnt = cnt + geq
        if 1 <= g <= n_graphs - 1:
            row = jnp.where(lane == g, n_edges - jnp.sum(geq), row)
    eb_ref[...] = jnp.clip(cnt - 1, 0, n_graphs - 1)
    ptrrow_ref[...] = row


def _make_edge_kernel(n_nodes, n_rel, hid, n_edges):
    K = 40
    n_chunks = n_edges // K
    n_my = n_chunks // NW
    e_my = n_my * K
    half = hid // 2
    n_grp = half // LANES
    e_pad = ((e_my + LANES - 1) // LANES) * LANES
    n_gv = e_pad // LANES

    mesh = plsc.VectorSubcoreMesh(core_axis_name="c", subcore_axis_name="s")

    @functools.partial(
        pl.kernel,
        out_type=(
            jax.ShapeDtypeStruct((n_edges, hid), jnp.float32),
            jax.ShapeDtypeStruct((n_edges,), jnp.int32),
            jax.ShapeDtypeStruct((n_edges,), jnp.int32),
        ),
        mesh=mesh,
        scratch_types=[
            pltpu.VMEM((n_nodes,), jnp.int32),
            pltpu.VMEM((e_pad,), jnp.int32),
            pltpu.VMEM((e_pad,), jnp.int32),
            pltpu.VMEM((e_my,), jnp.int32),
            pltpu.VMEM((e_pad,), jnp.int32),
            pltpu.VMEM((e_pad,), jnp.int32),
            pltpu.VMEM((K, half), jnp.int32),
            pltpu.VMEM((K, half), jnp.int32),
            pltpu.VMEM((K, half), jnp.int32),
            pltpu.VMEM((K, half), jnp.int32),
            pltpu.VMEM((K, half), jnp.int32),
            pltpu.VMEM((K, half), jnp.int32),
            pltpu.VMEM((K, half), jnp.int32),
            pltpu.VMEM((K, half), jnp.int32),
            pltpu.VMEM((K, half), jnp.int32),
            pltpu.VMEM((K, hid), jnp.float32),
            pltpu.VMEM((K, hid), jnp.float32),
            pltpu.VMEM((K, hid), jnp.float32),
            pltpu.SemaphoreType.DMA,
            pltpu.SemaphoreType.DMA,
            pltpu.SemaphoreType.DMA,
            pltpu.SemaphoreType.DMA,
            pltpu.SemaphoreType.DMA,
            pltpu.SemaphoreType.DMA,
        ],
        **_SC_PARAMS,
    )
    def edge_kernel(h_hbm, t_hbm, rp_hbm, src_hbm, dst_hbm, rel_hbm, ngid_hbm,
                    out_hbm, heads_hbm, tails_hbm,
                    ngid_v, src_all, dst_all, rel_all, hbuf_all, tbuf_all,
                    h0, h1, h2, t0, t1, t2, r0, r1, r2, o0, o1, o2,
                    semg0, semg1, semg2, semw0, semw1, semw2):
        h_rows = (h0, h1, h2)
        t_rows = (t0, t1, t2)
        r_rows = (r0, r1, r2)
        out_buf = (o0, o1, o2)
        semg = (semg0, semg1, semg2)
        semw = (semw0, semw1, semw2)
        wid = lax.axis_index("c") * NS + lax.axis_index("s")
        ebase = wid * e_my

        zeros16 = jnp.zeros((LANES,), jnp.int32)
        src_all[pl.ds(e_pad - LANES, LANES)] = zeros16
        dst_all[pl.ds(e_pad - LANES, LANES)] = zeros16
        stage = [
            pltpu.async_copy(src_hbm.at[pl.ds(ebase, e_my)],
                             src_all.at[pl.ds(0, e_my)], semw0),
            pltpu.async_copy(dst_hbm.at[pl.ds(ebase, e_my)],
                             dst_all.at[pl.ds(0, e_my)], semw0),
            pltpu.async_copy(rel_hbm.at[pl.ds(ebase, e_my)], rel_all, semw0),
            pltpu.async_copy(ngid_hbm, ngid_v, semw0),
        ]
        for cp in stage:
            cp.wait()

        def fire(c, b):
            lb = c * K
            pltpu.async_copy(h_hbm.at[src_all.at[pl.ds(lb, K)]],
                             h_rows[b], semg[b])
            pltpu.async_copy(t_hbm.at[dst_all.at[pl.ds(lb, K)]],
                             t_rows[b], semg[b])
            pltpu.async_copy(rp_hbm.at[rel_all.at[pl.ds(lb, K)]],
                             r_rows[b], semg[b])

        fire(0, 0)
        fire(1, 1)

        @plsc.parallel_loop(0, n_gv, unroll=4)
        def _gid(j):
            sl = pl.ds(j * LANES, LANES)
            hbuf_all[sl] = plsc.load_gather(ngid_v, [src_all[sl]])
            tbuf_all[sl] = plsc.load_gather(ngid_v, [dst_all[sl]])

        pltpu.async_copy(hbuf_all.at[pl.ds(0, e_my)],
                         heads_hbm.at[pl.ds(ebase, e_my)], semw0)
        pltpu.async_copy(tbuf_all.at[pl.ds(0, e_my)],
                         tails_hbm.at[pl.ds(ebase, e_my)], semw1)
        pltpu.make_async_copy(hbuf_all.at[pl.ds(0, e_my)],
                              heads_hbm.at[pl.ds(0, e_my)], semw0).wait()
        pltpu.make_async_copy(tbuf_all.at[pl.ds(0, e_my)],
                              tails_hbm.at[pl.ds(0, e_my)], semw1).wait()

        @pl.loop(0, n_my, step=3)
        def _trip(lc):
            for b in range(3):
                c = lc + b
                live = (c < n_my) if b else None

                def section():
                    @pl.when(c + 2 < n_my)
                    def _():
                        fire(c + 2, (b + 2) % 3)
                    for dst in (h_rows[b], t_rows[b], r_rows[b]):
                        pltpu.make_async_copy(
                            h_hbm.at[pl.ds(0, K)], dst, semg[b]).wait()
                    @pl.when(c >= 3)
                    def _():
                        pltpu.make_async_copy(
                            out_buf[b], out_hbm.at[pl.ds(0, K)],
                            semw[b]).wait()

                    mask_hi = jnp.int32(-65536)

                    @plsc.parallel_loop(0, K, unroll=4)
                    def _row(r):
                        for m in range(n_grp):
                            sl = pl.ds(m * LANES, LANES)
                            wh = h_rows[b][r, sl]
                            wt = t_rows[b][r, sl]
                            wr = r_rows[b][r, sl]
                            lo = (plsc.bitcast(wh << 16, jnp.float32)
                                  + plsc.bitcast(wt << 16, jnp.float32)
                                  + plsc.bitcast(wr << 16, jnp.float32))
                            hi = (plsc.bitcast(wh & mask_hi, jnp.float32)
                                  + plsc.bitcast(wt & mask_hi, jnp.float32)
                                  + plsc.bitcast(wr & mask_hi, jnp.float32))
                            out_buf[b][r, sl] = jnp.maximum(lo, 0.0)
                            out_buf[b][r, pl.ds(half + m * LANES, LANES)] = (
                                jnp.maximum(hi, 0.0))

                    pltpu.async_copy(
                        out_buf[b],
                        out_hbm.at[pl.ds(ebase + c * K, K)], semw[b])

                if live is None:
                    section()
                else:
                    pl.when(live)(section)

        for b in range(3):
            pltpu.make_async_copy(out_buf[b], out_hbm.at[pl.ds(0, K)],
                                  semw[b]).wait()

    return edge_kernel


def kernel(node_embedding_ids, node_global_ids, edge_index, edge_relations,
           question_emb, node_ptr, entity_table, relation_table, non_text_emb,
           W_ent, b_ent, W_rel, b_rel, W_q, b_q, W_edge, b_edge):
    n_nodes = node_embedding_ids.shape[0]
    n_edges = edge_relations.shape[0]
    hid = entity_table.shape[1]
    n_ent = entity_table.shape[0]
    n_rel = relation_table.shape[0]
    n_graphs = node_ptr.shape[0] - 1

    src = edge_index[0]
    dst = edge_index[1]

    W1 = W_edge[0 * hid:1 * hid]
    W2 = W_edge[1 * hid:2 * hid]
    W3 = W_edge[2 * hid:3 * hid]
    ids_col = node_embedding_ids.reshape(n_nodes, 1)
    src2d = src.reshape(n_edges // 128, 128)
    row2 = lambda v: v.reshape(1, hid)

    full = lambda shape: pl.BlockSpec(shape, lambda i: (0, 0))

    n_pad = ((n_nodes + NW * 80 - 1) // (NW * 80)) * (NW * 80)
    ids_pad = jnp.pad(node_embedding_ids, (0, n_pad - n_nodes))
    node_raw_pad = _make_entity_gather(n_ent, hid, n_pad)(entity_table, ids_pad)

    Rp, question_tokens, eb2d, ptr_row = pl.pallas_call(
        _tc_rel_body,
        grid=(1,),
        in_specs=[
            full((n_rel, hid)), full((hid, hid)), full((1, hid)),
            full((hid, hid)), full((1, hid)),
            full((question_emb.shape[0], hid)), full((hid, hid)), full((1, hid)),
            full((n_edges // 128, 128)),
            pl.BlockSpec(memory_space=pltpu.SMEM),
        ],
        out_specs=[
            full((n_rel, hid // 2)),
            full((question_emb.shape[0], hid)),
            full((n_edges // 128, 128)),
            full((1, 128)),
        ],
        out_shape=[
            jax.ShapeDtypeStruct((n_rel, hid // 2), jnp.int32),
            jax.ShapeDtypeStruct((question_emb.shape[0], hid), jnp.float32),
            jax.ShapeDtypeStruct((n_edges // 128, 128), jnp.int32),
            jax.ShapeDtypeStruct((1, 128), jnp.int32),
        ],
    )(relation_table, W_rel, row2(b_rel), W2, row2(b_edge),
      question_emb, W_q, row2(b_q), src2d, node_ptr)

    BLK = 2000
    n_blk = n_nodes // BLK
    blk_node = pl.BlockSpec((BLK, hid), lambda i: (i, 0))
    blk_pack = pl.BlockSpec((BLK, hid // 2), lambda i: (i, 0))

    node_tokens, H, T = pl.pallas_call(
        _tc_node_body,
        grid=(n_blk,),
        in_specs=[
            pl.BlockSpec((BLK, 1), lambda i: (i, 0)),
            blk_node,
            full((hid, hid)), full((1, hid)), full((1, hid)),
            full((hid, hid)), full((hid, hid)),
        ],
        out_specs=[blk_node, blk_pack, blk_pack],
        out_shape=[
            jax.ShapeDtypeStruct((n_nodes, hid), jnp.float32),
            jax.ShapeDtypeStruct((n_nodes, hid // 2), jnp.int32),
            jax.ShapeDtypeStruct((n_nodes, hid // 2), jnp.int32),
        ],
    )(ids_col, node_raw_pad, W_ent, row2(b_ent), row2(non_text_emb), W1, W3)

    edge_tokens, heads_global, tails_global = _make_edge_kernel(
        n_nodes, n_rel, hid, n_edges)(H, T, Rp, src, dst, edge_relations,
                                      node_global_ids)

    edge_batch = eb2d.reshape(n_edges)
    edge_ptr = ptr_row[0, :n_graphs + 1]
    return (edge_tokens, node_tokens, question_tokens, heads_global,
            tails_global, edge_batch, edge_ptr)
